# baseline (speedup 1.0000x reference)
import functools

import jax
import jax.numpy as jnp
from jax import lax
from jax.experimental import pallas as pl
from jax.experimental.pallas import tpu as pltpu
from jax.experimental.pallas import tpu_sc as plsc

_BETA = 0.8
_LAMBDA_P = 1.0
_LAMBDA_N = 0.1
_U_LOW = 0.02
_U_UP = 0.1
_W_LOW = 0.2
_W_UP = 0.8
_K = -(_W_UP - _W_LOW) / (_U_UP - _U_LOW)
_B = _W_LOW - _K * _U_UP

_N = 16 * 512 * 512
_COLS = 512
_ROWS = _N // _COLS
_NC = 2
_NS = 16
_NW = _NC * _NS

_S_ROWS = 2048
_RW = _S_ROWS // _NW
_CR = 32
_NCHUNK = _RW // _CR
_L = 16
_NSET = 4
_NACC = 6
_PROW = _NACC * _L

_BR = 64
_TGRID = (_ROWS - _S_ROWS) // _BR


def _stat_vals(r, labf, u):
    rp = r * labf
    rn = r - rp
    t = _K * u + _B
    w = jnp.maximum(t, _W_LOW)
    w = jnp.where(t > _W_UP, 1.0, w)
    return (labf, rp, rp * r, r, r * r, rn * w)


def _partials_body(r_hbm, lab_hbm, u_hbm, out_hbm,
                   rb0, lb0, ub0, rb1, lb1, ub1, stage, sem0, sem1):
    wid = lax.axis_index("s") * _NC + lax.axis_index("c")
    row0 = wid * _RW
    bufs = ((rb0, lb0, ub0, sem0), (rb1, lb1, ub1, sem1))

    def start(c):
        rb, lb, ub, sem = bufs[c % 2]
        r0 = row0 + c * _CR
        return (pltpu.async_copy(r_hbm.at[pl.ds(r0, _CR), :], rb, sem),
                pltpu.async_copy(lab_hbm.at[pl.ds(r0, _CR), :], lb, sem),
                pltpu.async_copy(u_hbm.at[pl.ds(r0, _CR), :], ub, sem))

    _BSTEPS = 4
    _BLK = _COLS // (_L * _BSTEPS)

    def chunk_accum(rb, lb, ub, acc):
        def body(i, acc):
            accl = list(acc)
            row = i // _BLK
            col0 = (i % _BLK) * (_L * _BSTEPS)
            for j in range(_BSTEPS):
                col = col0 + j * _L
                r = rb[row, pl.ds(col, _L)]
                labf = lb[row, pl.ds(col, _L)].astype(jnp.float32)
                u = ub[row, pl.ds(col, _L)]
                s = (j % _NSET) * _NACC
                vals = _stat_vals(r, labf, u)
                accl[s:s + _NACC] = [a + v for a, v in zip(accl[s:s + _NACC], vals)]
            return tuple(accl)
        return lax.fori_loop(0, _CR * _BLK, body, acc)

    zero = jnp.zeros((_L,), jnp.float32)
    acc = (zero,) * (_NACC * _NSET)

    handles = start(0)
    for c in range(_NCHUNK):
        for h in handles:
            h.wait()
        if c + 1 < _NCHUNK:
            nxt = start(c + 1)
        rb, lb, ub, _ = bufs[c % 2]
        acc = chunk_accum(rb, lb, ub, acc)
        if c + 1 < _NCHUNK:
            handles = nxt

    for k in range(_NACC):
        tot = acc[k]
        for s in range(1, _NSET):
            tot = tot + acc[s * _NACC + k]
        stage[pl.ds(k * _L, _L)] = tot
    pltpu.sync_copy(stage, out_hbm.at[wid])


_partials_kernel = functools.partial(
    pl.kernel,
    out_type=jax.ShapeDtypeStruct((_NW, _PROW), jnp.float32),
    mesh=plsc.VectorSubcoreMesh(core_axis_name="c", subcore_axis_name="s"),
    scratch_types=[
        pltpu.VMEM((_CR, _COLS), jnp.float32),
        pltpu.VMEM((_CR, _COLS), jnp.int32),
        pltpu.VMEM((_CR, _COLS), jnp.float32),
        pltpu.VMEM((_CR, _COLS), jnp.float32),
        pltpu.VMEM((_CR, _COLS), jnp.int32),
        pltpu.VMEM((_CR, _COLS), jnp.float32),
        pltpu.VMEM((_PROW,), jnp.float32),
        pltpu.SemaphoreType.DMA,
        pltpu.SemaphoreType.DMA,
    ],
)(_partials_body)


def _tc_body(r_ref, lab_ref, u_ref, o_ref, acc_ref):
    i = pl.program_id(0)

    @pl.when(i == 0)
    def _():
        acc_ref[...] = jnp.zeros_like(acc_ref)

    vals = _stat_vals(r_ref[...], lab_ref[...].astype(jnp.float32), u_ref[...])
    for k in range(_NACC):
        v = vals[k]
        slabs = [v[a * 8:(a + 1) * 8, :] for a in range(_BR // 8)]
        while len(slabs) > 1:
            slabs = [slabs[i] + slabs[i + 1] for i in range(0, len(slabs), 2)]
        acc_ref[k] += slabs[0]

    @pl.when(i == _TGRID - 1)
    def _():
        for k in range(_NACC):
            o_ref[k] = jnp.sum(acc_ref[k])


_tc_partial = pl.pallas_call(
    _tc_body,
    grid=(_TGRID,),
    in_specs=[
        pl.BlockSpec((_BR, _COLS), lambda i: (_S_ROWS // _BR + i, 0)),
        pl.BlockSpec((_BR, _COLS), lambda i: (_S_ROWS // _BR + i, 0)),
        pl.BlockSpec((_BR, _COLS), lambda i: (_S_ROWS // _BR + i, 0)),
    ],
    out_specs=pl.BlockSpec(memory_space=pltpu.SMEM),
    out_shape=jax.ShapeDtypeStruct((_NACC,), jnp.float32),
    scratch_shapes=[pltpu.VMEM((_NACC, 8, _COLS), jnp.float32)],
)


def _finish_body(p_ref, t_ref, o_ref):
    p = p_ref[...]
    s = [jnp.sum(p[:, k * _L:(k + 1) * _L]) + t_ref[k] for k in range(_NACC)]
    n_pos, s_rp, s_r2p, s_r, s_r2, s_rwn = s
    n_neg = _N - n_pos
    mean_p = s_rp / n_pos
    var_p = (s_r2p - s_rp * s_rp / n_pos) / (n_pos - 1.0)
    s_rn = s_r - s_rp
    s_r2n = s_r2 - s_r2p
    mean_n = s_rwn / n_neg
    var_n = (s_r2n - s_rn * s_rn / n_neg) / (n_neg - 1.0)
    loss = (jnp.maximum(_BETA - mean_p, 0.0) + _LAMBDA_N * var_p
            + mean_n + _LAMBDA_P * var_n)
    o_ref[0] = loss


_finish = pl.pallas_call(
    _finish_body,
    in_specs=[
        pl.BlockSpec(memory_space=pltpu.VMEM),
        pl.BlockSpec(memory_space=pltpu.SMEM),
    ],
    out_specs=pl.BlockSpec(memory_space=pltpu.SMEM),
    out_shape=jax.ShapeDtypeStruct((1,), jnp.float32),
)


def kernel(residues, pixel_level_labels, uncertainty_maps):
    r = residues.reshape(_ROWS, _COLS)
    lab = pixel_level_labels.reshape(_ROWS, _COLS).astype(jnp.int32)
    u = uncertainty_maps.reshape(_ROWS, _COLS)
    sc_parts = _partials_kernel(r, lab, u)
    tc_parts = _tc_partial(r, lab, u)
    return _finish(sc_parts, tc_parts)


# speedup vs baseline: 1.8386x; 1.8386x over previous
import functools

import jax
import jax.numpy as jnp
from jax import lax
from jax.experimental import pallas as pl
from jax.experimental.pallas import tpu as pltpu
from jax.experimental.pallas import tpu_sc as plsc

_BETA = 0.8
_LAMBDA_P = 1.0
_LAMBDA_N = 0.1
_U_LOW = 0.02
_U_UP = 0.1
_W_LOW = 0.2
_W_UP = 0.8
_K = -(_W_UP - _W_LOW) / (_U_UP - _U_LOW)
_B = _W_LOW - _K * _U_UP

_N = 16 * 512 * 512
_COLS = 512
_ROWS = _N // _COLS
_NC = 2
_NS = 16
_NW = _NC * _NS

_S_ROWS = 2048
_RW = _S_ROWS // _NW
_CR = 32
_NCHUNK = _RW // _CR
_L = 16
_NSET = 4
_NACC = 6
_PROW = _NACC * _L

_BR = 512
_TGRID = (_ROWS - _S_ROWS) // _BR


def _stat_vals(r, labf, u):
    rp = r * labf
    rn = r - rp
    t = _K * u + _B
    w = jnp.maximum(t, _W_LOW)
    w = jnp.where(t > _W_UP, 1.0, w)
    return (labf, rp, rp * r, r, r * r, rn * w)


def _partials_body(r_hbm, lab_hbm, u_hbm, out_hbm,
                   rb0, lb0, ub0, rb1, lb1, ub1, stage, sem0, sem1):
    wid = lax.axis_index("s") * _NC + lax.axis_index("c")
    row0 = wid * _RW
    bufs = ((rb0, lb0, ub0, sem0), (rb1, lb1, ub1, sem1))

    def start(c):
        rb, lb, ub, sem = bufs[c % 2]
        r0 = row0 + c * _CR
        return (pltpu.async_copy(r_hbm.at[pl.ds(r0, _CR), :], rb, sem),
                pltpu.async_copy(lab_hbm.at[pl.ds(r0, _CR), :], lb, sem),
                pltpu.async_copy(u_hbm.at[pl.ds(r0, _CR), :], ub, sem))

    _BSTEPS = 4
    _BLK = _COLS // (_L * _BSTEPS)

    def chunk_accum(rb, lb, ub, acc):
        def body(i, acc):
            accl = list(acc)
            row = i // _BLK
            col0 = (i % _BLK) * (_L * _BSTEPS)
            for j in range(_BSTEPS):
                col = col0 + j * _L
                r = rb[row, pl.ds(col, _L)]
                labf = lb[row, pl.ds(col, _L)].astype(jnp.float32)
                u = ub[row, pl.ds(col, _L)]
                s = (j % _NSET) * _NACC
                vals = _stat_vals(r, labf, u)
                accl[s:s + _NACC] = [a + v for a, v in zip(accl[s:s + _NACC], vals)]
            return tuple(accl)
        return lax.fori_loop(0, _CR * _BLK, body, acc)

    zero = jnp.zeros((_L,), jnp.float32)
    acc = (zero,) * (_NACC * _NSET)

    handles = start(0)
    for c in range(_NCHUNK):
        for h in handles:
            h.wait()
        if c + 1 < _NCHUNK:
            nxt = start(c + 1)
        rb, lb, ub, _ = bufs[c % 2]
        acc = chunk_accum(rb, lb, ub, acc)
        if c + 1 < _NCHUNK:
            handles = nxt

    for k in range(_NACC):
        tot = acc[k]
        for s in range(1, _NSET):
            tot = tot + acc[s * _NACC + k]
        stage[pl.ds(k * _L, _L)] = tot
    pltpu.sync_copy(stage, out_hbm.at[wid])


_partials_kernel = functools.partial(
    pl.kernel,
    out_type=jax.ShapeDtypeStruct((_NW, _PROW), jnp.float32),
    mesh=plsc.VectorSubcoreMesh(core_axis_name="c", subcore_axis_name="s"),
    scratch_types=[
        pltpu.VMEM((_CR, _COLS), jnp.float32),
        pltpu.VMEM((_CR, _COLS), jnp.int32),
        pltpu.VMEM((_CR, _COLS), jnp.float32),
        pltpu.VMEM((_CR, _COLS), jnp.float32),
        pltpu.VMEM((_CR, _COLS), jnp.int32),
        pltpu.VMEM((_CR, _COLS), jnp.float32),
        pltpu.VMEM((_PROW,), jnp.float32),
        pltpu.SemaphoreType.DMA,
        pltpu.SemaphoreType.DMA,
    ],
)(_partials_body)


def _tc_body(r_ref, lab_ref, u_ref, o_ref, acc_ref):
    i = pl.program_id(0)

    @pl.when(i == 0)
    def _():
        acc_ref[...] = jnp.zeros_like(acc_ref)

    vals = _stat_vals(r_ref[...], lab_ref[...].astype(jnp.float32), u_ref[...])
    for k in range(_NACC):
        v = vals[k]
        slabs = [v[a * 8:(a + 1) * 8, :] for a in range(_BR // 8)]
        while len(slabs) > 1:
            slabs = [slabs[i] + slabs[i + 1] for i in range(0, len(slabs), 2)]
        acc_ref[k] += slabs[0]

    @pl.when(i == _TGRID - 1)
    def _():
        for k in range(_NACC):
            o_ref[k] = jnp.sum(acc_ref[k])


_tc_partial = pl.pallas_call(
    _tc_body,
    grid=(_TGRID,),
    in_specs=[
        pl.BlockSpec((_BR, _COLS), lambda i: (_S_ROWS // _BR + i, 0)),
        pl.BlockSpec((_BR, _COLS), lambda i: (_S_ROWS // _BR + i, 0)),
        pl.BlockSpec((_BR, _COLS), lambda i: (_S_ROWS // _BR + i, 0)),
    ],
    out_specs=pl.BlockSpec(memory_space=pltpu.SMEM),
    out_shape=jax.ShapeDtypeStruct((_NACC,), jnp.float32),
    scratch_shapes=[pltpu.VMEM((_NACC, 8, _COLS), jnp.float32)],
)


def _finish_body(p_ref, t_ref, o_ref):
    p = p_ref[...]
    s = [jnp.sum(p[:, k * _L:(k + 1) * _L]) + t_ref[k] for k in range(_NACC)]
    n_pos, s_rp, s_r2p, s_r, s_r2, s_rwn = s
    n_neg = _N - n_pos
    mean_p = s_rp / n_pos
    var_p = (s_r2p - s_rp * s_rp / n_pos) / (n_pos - 1.0)
    s_rn = s_r - s_rp
    s_r2n = s_r2 - s_r2p
    mean_n = s_rwn / n_neg
    var_n = (s_r2n - s_rn * s_rn / n_neg) / (n_neg - 1.0)
    loss = (jnp.maximum(_BETA - mean_p, 0.0) + _LAMBDA_N * var_p
            + mean_n + _LAMBDA_P * var_n)
    o_ref[0] = loss


_finish = pl.pallas_call(
    _finish_body,
    in_specs=[
        pl.BlockSpec(memory_space=pltpu.VMEM),
        pl.BlockSpec(memory_space=pltpu.SMEM),
    ],
    out_specs=pl.BlockSpec(memory_space=pltpu.SMEM),
    out_shape=jax.ShapeDtypeStruct((1,), jnp.float32),
)


def kernel(residues, pixel_level_labels, uncertainty_maps):
    r = residues.reshape(_ROWS, _COLS)
    lab = pixel_level_labels.reshape(_ROWS, _COLS).astype(jnp.int32)
    u = uncertainty_maps.reshape(_ROWS, _COLS)
    sc_parts = _partials_kernel(r, lab, u)
    tc_parts = _tc_partial(r, lab, u)
    return _finish(sc_parts, tc_parts)
